# baseline (speedup 1.0000x reference)
import jax
import jax.numpy as jnp
from jax import lax
from jax.experimental import pallas as pl

TOP_K = 64
NHIDDEN = 16
D = 512
N = 1194
GROUPS = 6
GLEN = 199
BLK_M = 256
GRID_M = 5


def _norms_body(d1_ref, d2_ref, W_ref, s2_ref):
    f32 = jnp.float32
    diff = d1_ref[...] - d2_ref[...]
    E = jnp.dot(diff, W_ref[...], preferred_element_type=f32)
    Ee = E + 1e-6
    s2_ref[...] = jnp.sum(Ee * Ee, axis=1, keepdims=True)


def _topk_mlp_body(s2_ref, W1_ref, b1_ref, W2_ref, b2_ref, out_ref):
    f32 = jnp.float32
    s2row = s2_ref[...]

    eye = (lax.broadcasted_iota(jnp.int32, (GLEN, GLEN), 0)
           == lax.broadcasted_iota(jnp.int32, (GLEN, GLEN), 1)).astype(f32)
    subio = lax.broadcasted_iota(jnp.int32, (GLEN, GLEN), 0)
    lanio = lax.broadcasted_iota(jnp.int32, (GLEN, GLEN), 1)
    ones_col = jnp.ones((GLEN, 1), f32)
    kiof = lax.broadcasted_iota(jnp.int32, (GLEN, TOP_K), 1).astype(f32)

    xs_rows = []
    for g in range(GROUPS):
        rowg = s2row[:, g * GLEN:(g + 1) * GLEN]
        colg = lax.dot_general(
            eye, rowg, dimension_numbers=(((1,), (1,)), ((), ())),
            precision=lax.Precision.HIGHEST,
            preferred_element_type=f32)
        gt = rowg > colg
        tie = (rowg == colg) & (lanio < subio)
        cnt = gt.astype(f32) + tie.astype(f32)
        rank = lax.dot_general(
            cnt, ones_col, dimension_numbers=(((1,), (0,)), ((), ())),
            preferred_element_type=f32)
        oh = (rank == kiof).astype(f32)
        xs_rows.append(lax.dot_general(
            rowg, oh, dimension_numbers=(((1,), (0,)), ((), ())),
            precision=lax.Precision.HIGHEST,
            preferred_element_type=f32))

    xs = jnp.concatenate(xs_rows, axis=0)
    x = jnp.sqrt(xs)
    h = jnp.maximum(
        jnp.dot(x, W1_ref[...], precision=lax.Precision.HIGHEST,
                preferred_element_type=f32)
        + b1_ref[...], 0.0)
    out_ref[...] = (
        jnp.dot(h, W2_ref[...], precision=lax.Precision.HIGHEST,
                preferred_element_type=f32)
        + b2_ref[...])


def kernel(data1, data2, W_emb, b_emb, W1, b1, W2, b2):
    del b_emb
    s2col = pl.pallas_call(
        _norms_body,
        grid=(GRID_M,),
        in_specs=[
            pl.BlockSpec((BLK_M, D), lambda i: (i, 0)),
            pl.BlockSpec((BLK_M, D), lambda i: (i, 0)),
            pl.BlockSpec((D, D), lambda i: (0, 0)),
        ],
        out_specs=pl.BlockSpec((BLK_M, 1), lambda i: (i, 0)),
        out_shape=jax.ShapeDtypeStruct((N, 1), jnp.float32),
    )(data1, data2, W_emb)

    out = pl.pallas_call(
        _topk_mlp_body,
        out_shape=jax.ShapeDtypeStruct((GROUPS, 1), jnp.float32),
    )(s2col.reshape(1, N), W1, b1.reshape(1, NHIDDEN), W2, b2.reshape(1, 1))
    return out


# speedup vs baseline: 1.1430x; 1.1430x over previous
import jax
import jax.numpy as jnp
from jax import lax
from jax.experimental import pallas as pl
from jax.experimental.pallas import tpu as pltpu

TOP_K = 64
NHIDDEN = 16
D = 512
N = 1194
GROUPS = 6
GLEN = 199
BLK_M = 256
GRID_M = 5
HP = lax.Precision.HIGHEST
_DN = (((1,), (0,)), ((), ()))


def _split3(x):
    bf16 = jnp.bfloat16
    hi = x.astype(bf16)
    rem = x - hi.astype(jnp.float32)
    mid = rem.astype(bf16)
    lo = (rem - mid.astype(jnp.float32)).astype(bf16)
    return hi, mid, lo


def _body(d1_ref, d2_ref, W_ref, W1_ref, b1_ref, W2_ref, b2_ref, out_ref,
          s2_ref, Wh_ref, Wm_ref, Wl_ref):
    f32 = jnp.float32
    i = pl.program_id(0)

    @pl.when(i == 0)
    def _stage_w():
        Wh, Wm, Wl = _split3(W_ref[...])
        Wh_ref[...] = Wh
        Wm_ref[...] = Wm
        Wl_ref[...] = Wl

    diff = d1_ref[...] - d2_ref[...]
    dh, dm, dl = _split3(diff)
    dn = (((0,), (1,)), ((), ()))
    Et = (lax.dot_general(Wh_ref[...], dh, dn, preferred_element_type=f32)
          + (lax.dot_general(Wh_ref[...], dm, dn, preferred_element_type=f32)
             + lax.dot_general(Wm_ref[...], dh, dn,
                               preferred_element_type=f32)))
    Ee = Et + 1e-6
    s2_ref[pl.ds(i, 1), :] = jnp.sum(Ee * Ee, axis=0, keepdims=True)

    @pl.when(i == GRID_M - 1)
    def _epilogue():
        i32 = jnp.int32
        s2pad = jnp.concatenate(
            [s2_ref[j:j + 1, :] for j in range(GRID_M)], axis=1)
        s2row = s2pad[:, :N]

        v6 = jnp.concatenate(
            [s2row[:, g * GLEN:(g + 1) * GLEN] for g in range(GROUPS)],
            axis=0)
        eye = (lax.broadcasted_iota(i32, (GLEN, GLEN), 0)
               == lax.broadcasted_iota(i32, (GLEN, GLEN), 1)).astype(f32)
        colall = lax.dot_general(
            eye, v6, dimension_numbers=(((1,), (1,)), ((), ())),
            precision=HP, preferred_element_type=f32)

        gio = lax.broadcasted_iota(i32, (GROUPS, N), 0)
        tio = lax.broadcasted_iota(i32, (GROUPS, N), 1)
        expander = (tio // GLEN == gio).astype(f32)
        col_side = lax.dot_general(
            colall, expander, _DN, precision=HP,
            preferred_element_type=f32)

        jrow = lax.broadcasted_iota(i32, (1, N), 1) % GLEN
        icol = lax.broadcasted_iota(i32, (GLEN, 1), 0)
        gt = s2row > col_side
        tie = (s2row == col_side) & (jrow < icol)
        cnt = gt.astype(f32) + tie.astype(f32)

        blockones = (lax.broadcasted_iota(i32, (N, GROUPS), 0) // GLEN
                     == lax.broadcasted_iota(
                         i32, (N, GROUPS), 1)).astype(f32)
        rank_all = lax.dot_general(
            cnt, blockones, _DN, preferred_element_type=f32)

        exp64 = (lax.broadcasted_iota(i32, (GROUPS, GROUPS * TOP_K), 1)
                 // TOP_K
                 == lax.broadcasted_iota(
                     i32, (GROUPS, GROUPS * TOP_K), 0)).astype(f32)
        rank_exp = lax.dot_general(
            rank_all, exp64, _DN, preferred_element_type=f32)
        kio = (lax.broadcasted_iota(i32, (1, GROUPS * TOP_K), 1)
               % TOP_K).astype(f32)
        oh = (rank_exp == kio).astype(jnp.bfloat16)

        v6h = v6.astype(jnp.bfloat16)
        v6l = (v6 - v6h.astype(f32)).astype(jnp.bfloat16)
        X = (lax.dot_general(v6h, oh, _DN, preferred_element_type=f32)
             + lax.dot_general(v6l, oh, _DN, preferred_element_type=f32))
        gmask = (lax.broadcasted_iota(i32, (GROUPS, GROUPS * TOP_K), 1)
                 // TOP_K
                 == lax.broadcasted_iota(
                     i32, (GROUPS, GROUPS * TOP_K), 0)).astype(f32)
        Xm = X * gmask
        xs = Xm[:, 0:TOP_K]
        for b in range(1, GROUPS):
            xs = xs + Xm[:, b * TOP_K:(b + 1) * TOP_K]

        x = jnp.sqrt(xs)
        h = jnp.maximum(
            jnp.dot(x, W1_ref[...], precision=HP,
                    preferred_element_type=f32) + b1_ref[...], 0.0)
        out_ref[...] = (
            jnp.dot(h, W2_ref[...], precision=HP,
                    preferred_element_type=f32) + b2_ref[...])


def kernel(data1, data2, W_emb, b_emb, W1, b1, W2, b2):
    del b_emb
    out = pl.pallas_call(
        _body,
        grid=(GRID_M,),
        in_specs=[
            pl.BlockSpec((BLK_M, D), lambda i: (i, 0)),
            pl.BlockSpec((BLK_M, D), lambda i: (i, 0)),
            pl.BlockSpec((D, D), lambda i: (0, 0)),
            pl.BlockSpec((TOP_K, NHIDDEN), lambda i: (0, 0)),
            pl.BlockSpec((1, NHIDDEN), lambda i: (0, 0)),
            pl.BlockSpec((NHIDDEN, 1), lambda i: (0, 0)),
            pl.BlockSpec((1, 1), lambda i: (0, 0)),
        ],
        out_specs=pl.BlockSpec((GROUPS, 1), lambda i: (0, 0)),
        out_shape=jax.ShapeDtypeStruct((GROUPS, 1), jnp.float32),
        scratch_shapes=[
            pltpu.VMEM((GRID_M, BLK_M), jnp.float32),
            pltpu.VMEM((D, D), jnp.bfloat16),
            pltpu.VMEM((D, D), jnp.bfloat16),
            pltpu.VMEM((D, D), jnp.bfloat16),
        ],
    )(data1, data2, W_emb, W1, b1.reshape(1, NHIDDEN), W2, b2.reshape(1, 1))
    return out


# speedup vs baseline: 1.1887x; 1.0400x over previous
import jax
import jax.numpy as jnp
from jax import lax
from jax.experimental import pallas as pl
from jax.experimental.pallas import tpu as pltpu

TOP_K = 64
NHIDDEN = 16
D = 512
N = 1194
GROUPS = 6
GLEN = 199
BLK_M = 256
GRID_M = 5
HP = lax.Precision.HIGHEST
_DN = (((1,), (0,)), ((), ()))


def _body(d1_ref, d2_ref, W_ref, W1_ref, b1_ref, W2_ref, b2_ref, out_ref,
          s2_ref, Wh_ref):
    f32 = jnp.float32
    bf16 = jnp.bfloat16
    i = pl.program_id(0)

    @pl.when(i == 0)
    def _stage_w():
        Wh_ref[...] = W_ref[...].astype(bf16)

    dn = (((0,), (1,)), ((), ()))
    E1 = lax.dot_general(Wh_ref[...], d1_ref[...].astype(bf16), dn,
                         preferred_element_type=f32)
    E2 = lax.dot_general(Wh_ref[...], d2_ref[...].astype(bf16), dn,
                         preferred_element_type=f32)
    Ee = (E1 - E2) + 1e-6
    s2_ref[pl.ds(i, 1), :] = jnp.sum(Ee * Ee, axis=0, keepdims=True)

    @pl.when(i == GRID_M - 1)
    def _epilogue():
        i32 = jnp.int32
        s2pad = jnp.concatenate(
            [s2_ref[j:j + 1, :] for j in range(GRID_M)], axis=1)
        s2row = s2pad[:, :N]

        v6 = jnp.concatenate(
            [s2row[:, g * GLEN:(g + 1) * GLEN] for g in range(GROUPS)],
            axis=0)
        eye = (lax.broadcasted_iota(i32, (GLEN, GLEN), 0)
               == lax.broadcasted_iota(i32, (GLEN, GLEN), 1)).astype(f32)
        colall = lax.dot_general(
            eye, v6, dimension_numbers=(((1,), (1,)), ((), ())),
            precision=HP, preferred_element_type=f32)

        gio = lax.broadcasted_iota(i32, (GROUPS, N), 0)
        tio = lax.broadcasted_iota(i32, (GROUPS, N), 1)
        expander = (tio // GLEN == gio).astype(f32)
        col_side = lax.dot_general(
            colall, expander, _DN, precision=HP,
            preferred_element_type=f32)

        jrow = lax.broadcasted_iota(i32, (1, N), 1) % GLEN
        icol = lax.broadcasted_iota(i32, (GLEN, 1), 0)
        gt = s2row > col_side
        tie = (s2row == col_side) & (jrow < icol)
        cnt = gt.astype(f32) + tie.astype(f32)

        blockones = (lax.broadcasted_iota(i32, (N, GROUPS), 0) // GLEN
                     == lax.broadcasted_iota(
                         i32, (N, GROUPS), 1)).astype(f32)
        rank_all = lax.dot_general(
            cnt, blockones, _DN, preferred_element_type=f32)

        exp64 = (lax.broadcasted_iota(i32, (GROUPS, GROUPS * TOP_K), 1)
                 // TOP_K
                 == lax.broadcasted_iota(
                     i32, (GROUPS, GROUPS * TOP_K), 0)).astype(f32)
        rank_exp = lax.dot_general(
            rank_all, exp64, _DN, preferred_element_type=f32)
        kio = (lax.broadcasted_iota(i32, (1, GROUPS * TOP_K), 1)
               % TOP_K).astype(f32)
        oh = (rank_exp == kio).astype(jnp.bfloat16)

        v6h = v6.astype(jnp.bfloat16)
        v6l = (v6 - v6h.astype(f32)).astype(jnp.bfloat16)
        X = (lax.dot_general(v6h, oh, _DN, preferred_element_type=f32)
             + lax.dot_general(v6l, oh, _DN, preferred_element_type=f32))
        gmask = (lax.broadcasted_iota(i32, (GROUPS, GROUPS * TOP_K), 1)
                 // TOP_K
                 == lax.broadcasted_iota(
                     i32, (GROUPS, GROUPS * TOP_K), 0)).astype(f32)
        Xm = X * gmask
        xs = Xm[:, 0:TOP_K]
        for b in range(1, GROUPS):
            xs = xs + Xm[:, b * TOP_K:(b + 1) * TOP_K]

        x = jnp.sqrt(xs)
        bf16 = jnp.bfloat16
        h = jnp.maximum(
            lax.dot_general(x.astype(bf16), W1_ref[...].astype(bf16), _DN,
                            preferred_element_type=f32) + b1_ref[...], 0.0)
        out_ref[...] = (
            lax.dot_general(h.astype(bf16), W2_ref[...].astype(bf16), _DN,
                            preferred_element_type=f32) + b2_ref[...])


def kernel(data1, data2, W_emb, b_emb, W1, b1, W2, b2):
    del b_emb
    out = pl.pallas_call(
        _body,
        grid=(GRID_M,),
        in_specs=[
            pl.BlockSpec((BLK_M, D), lambda i: (i, 0)),
            pl.BlockSpec((BLK_M, D), lambda i: (i, 0)),
            pl.BlockSpec((D, D), lambda i: (0, 0)),
            pl.BlockSpec((TOP_K, NHIDDEN), lambda i: (0, 0)),
            pl.BlockSpec((1, NHIDDEN), lambda i: (0, 0)),
            pl.BlockSpec((NHIDDEN, 1), lambda i: (0, 0)),
            pl.BlockSpec((1, 1), lambda i: (0, 0)),
        ],
        out_specs=pl.BlockSpec((GROUPS, 1), lambda i: (0, 0)),
        out_shape=jax.ShapeDtypeStruct((GROUPS, 1), jnp.float32),
        scratch_shapes=[
            pltpu.VMEM((GRID_M, BLK_M), jnp.float32),
            pltpu.VMEM((D, D), jnp.bfloat16),
        ],
    )(data1, data2, W_emb, W1, b1.reshape(1, NHIDDEN), W2, b2.reshape(1, 1))
    return out
